# baseline (speedup 1.0000x reference)
import functools

import numpy as np
import jax
import jax.numpy as jnp
from jax import lax
from jax.experimental import pallas as pl
from jax.experimental.pallas import tpu as pltpu
from jax.experimental.pallas import tpu_sc as plsc

B, F, L, V, D = 4096, 26, 50, 100000, 64
NC, NS = 2, 16
NW = NC * NS
BPW = B // NW
SP_STRIPS = BPW * F // 128
SQ_STRIPS = BPW * L // 128
SQ_PAD = 56
NQ = D // 16
NBUF = 4


@functools.lru_cache(maxsize=1)
def _qdst_array():
    j = np.arange(B * L, dtype=np.int32)
    b = j // L
    qdst = (((b // BPW) // NC) * BPW + (b % BPW)).astype(np.int32).reshape(NW, SQ_STRIPS, 128)
    qdst = np.pad(qdst, ((0, 0), (0, SQ_PAD - SQ_STRIPS), (0, 0))).reshape(NW * SQ_PAD, 128)
    return qdst


_MESH = plsc.VectorSubcoreMesh(core_axis_name="c", subcore_axis_name="s")
_PARAMS = pltpu.CompilerParams(use_tc_tiling_on_sc=False)


def _body(idx1, hist1, tsp, tsq, qdst2, sp_out, pool_out,
          sidx, hidx, qdstv, r0, r1, r2, r3, acc, shacc,
          g0, g1, g2, g3, s0, s1, s2, s3):
    c = lax.axis_index("c")
    s = lax.axis_index("s")
    w = s * NC + c
    rows = (r0, r1, r2, r3)
    gs = (g0, g1, g2, g3)
    ss = (s0, s1, s2, s3)

    def drain(sem, dst):
        pltpu.make_async_copy(tsp.at[pl.ds(0, 128)], dst, sem).wait()

    st0 = pltpu.async_copy(
        idx1.at[pl.ds(pl.multiple_of(w * (SP_STRIPS * 128), 128), SP_STRIPS * 128)],
        sidx, g0)
    st1 = pltpu.async_copy(
        hist1.at[pl.ds(pl.multiple_of(w * (SQ_STRIPS * 128), 128), SQ_STRIPS * 128)],
        hidx, g1)
    st2 = pltpu.async_copy(qdst2.at[pl.ds(pl.multiple_of(w * SQ_PAD, 8), SQ_PAD)], qdstv, g2)

    def _zero(r, carry):
        for q in range(NQ):
            acc[r, pl.ds(q * 16, 16)] = jnp.zeros((16,), jnp.float32)
        return carry
    lax.fori_loop(0, BPW, _zero, 0)
    st0.wait()
    st1.wait()
    st2.wait()
    pltpu.sync_copy(acc, shacc.at[pl.ds(s * BPW, BPW)])

    def ring_phase(n, gath, store):
        for t in range(NBUF - 1):
            gath(t, rows[t], gs[t])

        def _group(p, carry):
            for j in range(NBUF):
                tt = NBUF * p + j
                drain(gs[j], rows[j])
                store(tt, rows[j], ss[j])
                jn = (j + NBUF - 1) % NBUF

                @pl.when(tt + NBUF - 1 < n)
                def _():
                    @pl.when(tt >= 1)
                    def _():
                        drain(ss[jn], rows[jn])
                    gath(tt + NBUF - 1, rows[jn], gs[jn])
            return carry
        lax.fori_loop(0, n // NBUF, _group, 0)
        for j in range(n % NBUF):
            drain(gs[j], rows[j])
            store((n // NBUF) * NBUF + j, rows[j], ss[j])
        for j in range(NBUF):
            drain(ss[j], rows[j])

    obase = w * (SP_STRIPS * 128)

    def sp_gath(t, dst, sem):
        gi = sidx.at[pl.ds(pl.multiple_of(t * 128, 128), 128)]
        pltpu.async_copy(tsp.at[gi], dst, sem)

    def sp_store(t, src, sem):
        pltpu.async_copy(
            src, sp_out.at[pl.ds(pl.multiple_of(obase + t * 128, 128), 128)], sem)

    ring_phase(SP_STRIPS, sp_gath, sp_store)

    def sq_gath(t, dst, sem):
        gi = hidx.at[pl.ds(pl.multiple_of(t * 128, 128), 128)]
        pltpu.async_copy(tsq.at[gi], dst, sem)

    def sq_store(t, src, sem):
        pltpu.async_copy(src, shacc.at[qdstv.at[t]], sem, add=True)

    ring_phase(SQ_STRIPS, sq_gath, sq_store)

    pltpu.sync_copy(shacc.at[pl.ds(s * BPW, BPW)], acc)

    def _scale(r, carry):
        for q in range(NQ):
            acc[r, pl.ds(q * 16, 16)] = acc[r, pl.ds(q * 16, 16)] * (1.0 / L)
        return carry
    lax.fori_loop(0, BPW, _scale, 0)
    pltpu.sync_copy(acc, pool_out.at[pl.ds(pl.multiple_of(w * BPW, 128), BPW)])


_sc_call = functools.partial(
    pl.kernel,
    out_type=(
        jax.ShapeDtypeStruct((B * F, D), jnp.float32),
        jax.ShapeDtypeStruct((B, D), jnp.float32),
    ),
    mesh=_MESH,
    compiler_params=_PARAMS,
    scratch_types=[
        pltpu.VMEM((SP_STRIPS * 128,), jnp.int32),
        pltpu.VMEM((SQ_STRIPS * 128,), jnp.int32),
        pltpu.VMEM((SQ_PAD, 128), jnp.int32),
        pltpu.VMEM((128, D), jnp.float32),
        pltpu.VMEM((128, D), jnp.float32),
        pltpu.VMEM((128, D), jnp.float32),
        pltpu.VMEM((128, D), jnp.float32),
        pltpu.VMEM((BPW, D), jnp.float32),
        pltpu.VMEM_SHARED((NS * BPW, D), jnp.float32),
        pltpu.SemaphoreType.DMA,
        pltpu.SemaphoreType.DMA,
        pltpu.SemaphoreType.DMA,
        pltpu.SemaphoreType.DMA,
        pltpu.SemaphoreType.DMA,
        pltpu.SemaphoreType.DMA,
        pltpu.SemaphoreType.DMA,
        pltpu.SemaphoreType.DMA,
    ],
)(_body)


def kernel(indices, hist, table_sparse, table_seq):
    idx1 = indices.astype(jnp.int32).reshape(-1)
    hist1 = hist.astype(jnp.int32).reshape(-1)
    sp, pool = _sc_call(idx1, hist1, table_sparse, table_seq,
                        jnp.asarray(_qdst_array()))
    return jnp.concatenate([sp.reshape(B, F * D), pool], axis=-1)


# speedup vs baseline: 1.0540x; 1.0540x over previous
import functools

import numpy as np
import jax
import jax.numpy as jnp
from jax import lax
from jax.experimental import pallas as pl
from jax.experimental.pallas import tpu as pltpu
from jax.experimental.pallas import tpu_sc as plsc

B, F, L, V, D = 4096, 26, 50, 100000, 64
NC, NS = 2, 16
NW = NC * NS
BPW = B // NW
BH = B // 2
BPWH = BH // NW
SP_STRIPS = BPWH * F // 128
SQ_STRIPS = BPW * L // 128
SQ_PAD = 56
NQ = D // 16


@functools.lru_cache(maxsize=1)
def _qdst_array():
    j = np.arange(B * L, dtype=np.int32)
    b = j // L
    qdst = (((b // BPW) // NC) * BPW + (b % BPW)).astype(np.int32).reshape(NW, SQ_STRIPS, 128)
    qdst = np.pad(qdst, ((0, 0), (0, SQ_PAD - SQ_STRIPS), (0, 0))).reshape(NW * SQ_PAD, 128)
    return qdst


@functools.lru_cache(maxsize=1)
def _scale_array():
    scale = np.ones(((F + 1) * D,), np.float32)
    scale[F * D:] = 1.0 / L
    return scale


_MESH = plsc.VectorSubcoreMesh(core_axis_name="c", subcore_axis_name="s")
_PARAMS = pltpu.CompilerParams(use_tc_tiling_on_sc=False)


def _make_sparse_body(sample_base):
    def _sparse_body(idx1, tsp, out, sidx, rows0, rows1, g0, g1, s0, s1):
        c = lax.axis_index("c")
        s = lax.axis_index("s")
        w = s * NC + c
        ibase = sample_base * F + w * (SP_STRIPS * 128)
        obase = w * (SP_STRIPS * 128)

        def drain(sem, dst):
            pltpu.make_async_copy(tsp.at[pl.ds(0, 128)], dst, sem).wait()

        pltpu.sync_copy(
            idx1.at[pl.ds(pl.multiple_of(ibase, 128), SP_STRIPS * 128)], sidx)

        def gath(t, dst, sem):
            gi = sidx.at[pl.ds(pl.multiple_of(t * 128, 128), 128)]
            pltpu.async_copy(tsp.at[gi], dst, sem)

        def store(t, src, sem):
            pltpu.async_copy(
                src, out.at[pl.ds(pl.multiple_of(obase + t * 128, 128), 128)], sem)

        gath(0, rows0, g0)

        def _pair(p, carry):
            t0 = 2 * p

            @pl.when(p > 0)
            def _():
                drain(s1, rows1)
            gath(t0 + 1, rows1, g1)
            drain(g0, rows0)
            store(t0, rows0, s0)

            @pl.when(p < SP_STRIPS // 2 - 1)
            def _():
                drain(s0, rows0)
                gath(t0 + 2, rows0, g0)
            drain(g1, rows1)
            store(t0 + 1, rows1, s1)
            return carry
        lax.fori_loop(0, SP_STRIPS // 2, _pair, 0)
        drain(s1, rows1)
        gath(SP_STRIPS - 1, rows1, g1)
        drain(s0, rows0)
        drain(g1, rows1)
        store(SP_STRIPS - 1, rows1, s1)
        drain(s1, rows1)
    return _sparse_body


def _seq_body(hist1, tsq, qdst2, out, hidx, qdstv, rows0, rows1, acc,
              shacc, g0, g1, s0, s1):
    c = lax.axis_index("c")
    s = lax.axis_index("s")
    w = s * NC + c

    def drain(sem, dst):
        pltpu.make_async_copy(tsq.at[pl.ds(0, 128)], dst, sem).wait()

    st0 = pltpu.async_copy(
        hist1.at[pl.ds(pl.multiple_of(w * (SQ_STRIPS * 128), 128), SQ_STRIPS * 128)],
        hidx, g0)
    st1 = pltpu.async_copy(qdst2.at[pl.ds(pl.multiple_of(w * SQ_PAD, 8), SQ_PAD)], qdstv, g1)

    def _zero(r, carry):
        for q in range(NQ):
            acc[r, pl.ds(q * 16, 16)] = jnp.zeros((16,), jnp.float32)
        return carry
    lax.fori_loop(0, BPW, _zero, 0)
    st0.wait()
    st1.wait()
    pltpu.sync_copy(acc, shacc.at[pl.ds(s * BPW, BPW)])

    def gath(t, dst, sem):
        gi = hidx.at[pl.ds(pl.multiple_of(t * 128, 128), 128)]
        pltpu.async_copy(tsq.at[gi], dst, sem)

    gath(0, rows0, g0)

    def _pair(p, carry):
        t0 = 2 * p

        @pl.when(p > 0)
        def _():
            drain(s1, rows1)
        gath(t0 + 1, rows1, g1)
        drain(g0, rows0)
        pltpu.async_copy(rows0, shacc.at[qdstv.at[t0]], s0, add=True)

        @pl.when(p < SQ_STRIPS // 2 - 1)
        def _():
            drain(s0, rows0)
            gath(t0 + 2, rows0, g0)
        drain(g1, rows1)
        pltpu.async_copy(rows1, shacc.at[qdstv.at[t0 + 1]], s1, add=True)
        return carry
    lax.fori_loop(0, SQ_STRIPS // 2, _pair, 0)
    drain(s0, rows0)
    drain(s1, rows1)

    pltpu.sync_copy(shacc.at[pl.ds(s * BPW, BPW)], acc)
    pltpu.sync_copy(acc, out.at[pl.ds(pl.multiple_of(w * BPW, 128), BPW)])


def _make_sparse_call(sample_base):
    return functools.partial(
        pl.kernel,
        out_type=jax.ShapeDtypeStruct((BH * F, D), jnp.float32),
        mesh=_MESH,
        compiler_params=_PARAMS,
        scratch_types=[
            pltpu.VMEM((SP_STRIPS * 128,), jnp.int32),
            pltpu.VMEM((128, D), jnp.float32),
            pltpu.VMEM((128, D), jnp.float32),
            pltpu.SemaphoreType.DMA,
            pltpu.SemaphoreType.DMA,
            pltpu.SemaphoreType.DMA,
            pltpu.SemaphoreType.DMA,
        ],
    )(_make_sparse_body(sample_base))


_sparse_call_0 = _make_sparse_call(0)
_sparse_call_1 = _make_sparse_call(BH)

_seq_call = functools.partial(
    pl.kernel,
    out_type=jax.ShapeDtypeStruct((B, D), jnp.float32),
    mesh=_MESH,
    compiler_params=_PARAMS,
    scratch_types=[
        pltpu.VMEM((SQ_STRIPS * 128,), jnp.int32),
        pltpu.VMEM((SQ_PAD, 128), jnp.int32),
        pltpu.VMEM((128, D), jnp.float32),
        pltpu.VMEM((128, D), jnp.float32),
        pltpu.VMEM((BPW, D), jnp.float32),
        pltpu.VMEM_SHARED((NS * BPW, D), jnp.float32),
        pltpu.SemaphoreType.DMA,
        pltpu.SemaphoreType.DMA,
        pltpu.SemaphoreType.DMA,
        pltpu.SemaphoreType.DMA,
    ],
)(_seq_body)


def kernel(indices, hist, table_sparse, table_seq):
    idx1 = indices.astype(jnp.int32).reshape(-1)
    hist1 = hist.astype(jnp.int32).reshape(-1)
    sp1 = _sparse_call_0(idx1, table_sparse)
    sp2 = _sparse_call_1(idx1, table_sparse)
    pool = _seq_call(hist1, table_seq, jnp.asarray(_qdst_array()))
    sp = jnp.concatenate(
        [sp1.reshape(BH, F * D), sp2.reshape(BH, F * D)], axis=0)
    out = jnp.concatenate([sp, pool], axis=-1)
    return out * jnp.asarray(_scale_array())[None, :]


# speedup vs baseline: 1.0929x; 1.0368x over previous
import functools

import numpy as np
import jax
import jax.numpy as jnp
from jax import lax
from jax.experimental import pallas as pl
from jax.experimental.pallas import tpu as pltpu
from jax.experimental.pallas import tpu_sc as plsc

B, F, L, V, D = 4096, 26, 50, 100000, 64
NC, NS = 2, 16
NW = NC * NS
BPW = B // NW
BH = B // 2
BPWH = BH // NW
SP_STRIPS = BPWH * F // 128
SQ_STRIPS = BPW * L // 128
SQ_PAD = 56
NQ = D // 16


@functools.lru_cache(maxsize=1)
def _qdst_array():
    j = np.arange(B * L, dtype=np.int32)
    b = j // L
    qdst = (((b // BPW) // NC) * BPW + (b % BPW)).astype(np.int32).reshape(NW, SQ_STRIPS, 128)
    qdst = np.pad(qdst, ((0, 0), (0, SQ_PAD - SQ_STRIPS), (0, 0))).reshape(NW * SQ_PAD, 128)
    return qdst


_MESH = plsc.VectorSubcoreMesh(core_axis_name="c", subcore_axis_name="s")
_PARAMS = pltpu.CompilerParams(use_tc_tiling_on_sc=False)


def _make_sparse_body(sample_base):
    def _sparse_body(idx1, tsp, out, sidx, rows0, rows1, g0, g1, s0, s1):
        c = lax.axis_index("c")
        s = lax.axis_index("s")
        w = s * NC + c
        ibase = sample_base * F + w * (SP_STRIPS * 128)
        obase = w * (SP_STRIPS * 128)

        def drain(sem, dst):
            pltpu.make_async_copy(tsp.at[pl.ds(0, 128)], dst, sem).wait()

        pltpu.sync_copy(
            idx1.at[pl.ds(pl.multiple_of(ibase, 128), SP_STRIPS * 128)], sidx)

        def gath(t, dst, sem):
            gi = sidx.at[pl.ds(pl.multiple_of(t * 128, 128), 128)]
            pltpu.async_copy(tsp.at[gi], dst, sem)

        def store(t, src, sem):
            pltpu.async_copy(
                src, out.at[pl.ds(pl.multiple_of(obase + t * 128, 128), 128)], sem)

        gath(0, rows0, g0)

        def _pair(p, carry):
            t0 = 2 * p

            @pl.when(p > 0)
            def _():
                drain(s1, rows1)
            gath(t0 + 1, rows1, g1)
            drain(g0, rows0)
            store(t0, rows0, s0)

            @pl.when(p < SP_STRIPS // 2 - 1)
            def _():
                drain(s0, rows0)
                gath(t0 + 2, rows0, g0)
            drain(g1, rows1)
            store(t0 + 1, rows1, s1)
            return carry
        lax.fori_loop(0, SP_STRIPS // 2, _pair, 0)
        drain(s1, rows1)
        gath(SP_STRIPS - 1, rows1, g1)
        drain(s0, rows0)
        drain(g1, rows1)
        store(SP_STRIPS - 1, rows1, s1)
        drain(s1, rows1)
    return _sparse_body


def _seq_body(hist1, tsq, qdst2, out, hidx, qdstv, rows0, rows1, acc,
              shacc, g0, g1, s0, s1):
    c = lax.axis_index("c")
    s = lax.axis_index("s")
    w = s * NC + c

    def drain(sem, dst):
        pltpu.make_async_copy(tsq.at[pl.ds(0, 128)], dst, sem).wait()

    st0 = pltpu.async_copy(
        hist1.at[pl.ds(pl.multiple_of(w * (SQ_STRIPS * 128), 128), SQ_STRIPS * 128)],
        hidx, g0)
    st1 = pltpu.async_copy(qdst2.at[pl.ds(pl.multiple_of(w * SQ_PAD, 8), SQ_PAD)], qdstv, g1)

    def _zero(r, carry):
        for q in range(NQ):
            acc[r, pl.ds(q * 16, 16)] = jnp.zeros((16,), jnp.float32)
        return carry
    lax.fori_loop(0, BPW, _zero, 0)
    st0.wait()
    st1.wait()
    pltpu.sync_copy(acc, shacc.at[pl.ds(s * BPW, BPW)])

    def gath(t, dst, sem):
        gi = hidx.at[pl.ds(pl.multiple_of(t * 128, 128), 128)]
        pltpu.async_copy(tsq.at[gi], dst, sem)

    gath(0, rows0, g0)

    def _pair(p, carry):
        t0 = 2 * p

        @pl.when(p > 0)
        def _():
            drain(s1, rows1)
        gath(t0 + 1, rows1, g1)
        drain(g0, rows0)
        pltpu.async_copy(rows0, shacc.at[qdstv.at[t0]], s0, add=True)

        @pl.when(p < SQ_STRIPS // 2 - 1)
        def _():
            drain(s0, rows0)
            gath(t0 + 2, rows0, g0)
        drain(g1, rows1)
        pltpu.async_copy(rows1, shacc.at[qdstv.at[t0 + 1]], s1, add=True)
        return carry
    lax.fori_loop(0, SQ_STRIPS // 2, _pair, 0)
    drain(s0, rows0)
    drain(s1, rows1)

    pltpu.sync_copy(shacc.at[pl.ds(s * BPW, BPW)], acc)

    def _scale(r, carry):
        for q in range(NQ):
            acc[r, pl.ds(q * 16, 16)] = acc[r, pl.ds(q * 16, 16)] * (1.0 / L)
        return carry
    lax.fori_loop(0, BPW, _scale, 0)
    pltpu.sync_copy(acc, out.at[pl.ds(pl.multiple_of(w * BPW, 128), BPW)])


def _make_sparse_call(sample_base):
    return functools.partial(
        pl.kernel,
        out_type=jax.ShapeDtypeStruct((BH * F, D), jnp.float32),
        mesh=_MESH,
        compiler_params=_PARAMS,
        scratch_types=[
            pltpu.VMEM((SP_STRIPS * 128,), jnp.int32),
            pltpu.VMEM((128, D), jnp.float32),
            pltpu.VMEM((128, D), jnp.float32),
            pltpu.SemaphoreType.DMA,
            pltpu.SemaphoreType.DMA,
            pltpu.SemaphoreType.DMA,
            pltpu.SemaphoreType.DMA,
        ],
    )(_make_sparse_body(sample_base))


_sparse_call_0 = _make_sparse_call(0)
_sparse_call_1 = _make_sparse_call(BH)

_seq_call = functools.partial(
    pl.kernel,
    out_type=jax.ShapeDtypeStruct((B, D), jnp.float32),
    mesh=_MESH,
    compiler_params=_PARAMS,
    scratch_types=[
        pltpu.VMEM((SQ_STRIPS * 128,), jnp.int32),
        pltpu.VMEM((SQ_PAD, 128), jnp.int32),
        pltpu.VMEM((128, D), jnp.float32),
        pltpu.VMEM((128, D), jnp.float32),
        pltpu.VMEM((BPW, D), jnp.float32),
        pltpu.VMEM_SHARED((NS * BPW, D), jnp.float32),
        pltpu.SemaphoreType.DMA,
        pltpu.SemaphoreType.DMA,
        pltpu.SemaphoreType.DMA,
        pltpu.SemaphoreType.DMA,
    ],
)(_seq_body)


def kernel(indices, hist, table_sparse, table_seq):
    idx1 = indices.astype(jnp.int32).reshape(-1)
    hist1 = hist.astype(jnp.int32).reshape(-1)
    sp1 = _sparse_call_0(idx1, table_sparse)
    sp2 = _sparse_call_1(idx1, table_sparse)
    pool = _seq_call(hist1, table_seq, jnp.asarray(_qdst_array()))
    sp = jnp.concatenate(
        [sp1.reshape(BH, F * D), sp2.reshape(BH, F * D)], axis=0)
    return jnp.concatenate([sp, pool], axis=-1)


# speedup vs baseline: 1.1197x; 1.0246x over previous
import functools

import numpy as np
import jax
import jax.numpy as jnp
from jax import lax
from jax.experimental import pallas as pl
from jax.experimental.pallas import tpu as pltpu
from jax.experimental.pallas import tpu_sc as plsc

B, F, L, V, D = 4096, 26, 50, 100000, 64
NC, NS = 2, 16
NW = NC * NS
BPW = B // NW
BH = B // 2
BPWH = BH // NW
SP_STRIPS = BPWH * F // 128
SQ_STRIPS = BPW * L // 128
SQ_PAD = 56
NQ = D // 16
NBUF = 4


def _ring_phase(n, rows, gs, ss, gath, store, drain):
    for t in range(NBUF - 1):
        gath(t, rows[t], gs[t])

    def _group(p, carry):
        for j in range(NBUF):
            tt = NBUF * p + j
            drain(gs[j], rows[j])
            store(tt, rows[j], ss[j])
            jn = (j + NBUF - 1) % NBUF

            @pl.when(tt + NBUF - 1 < n)
            def _():
                @pl.when(tt >= 1)
                def _():
                    drain(ss[jn], rows[jn])
                gath(tt + NBUF - 1, rows[jn], gs[jn])
        return carry
    lax.fori_loop(0, n // NBUF, _group, 0)
    for j in range(n % NBUF):
        drain(gs[j], rows[j])
        store((n // NBUF) * NBUF + j, rows[j], ss[j])
    for j in range(NBUF):
        drain(ss[j], rows[j])


@functools.lru_cache(maxsize=1)
def _qdst_array():
    j = np.arange(B * L, dtype=np.int32)
    b = j // L
    qdst = (((b // BPW) // NC) * BPW + (b % BPW)).astype(np.int32).reshape(NW, SQ_STRIPS, 128)
    qdst = np.pad(qdst, ((0, 0), (0, SQ_PAD - SQ_STRIPS), (0, 0))).reshape(NW * SQ_PAD, 128)
    return qdst


_MESH = plsc.VectorSubcoreMesh(core_axis_name="c", subcore_axis_name="s")
_PARAMS = pltpu.CompilerParams(use_tc_tiling_on_sc=False)


def _make_sparse_body(sample_base):
    def _sparse_body(idx1, tsp, out, sidx, r0, r1, r2, r3,
                     g0, g1, g2, g3, s0, s1, s2, s3):
        c = lax.axis_index("c")
        s = lax.axis_index("s")
        w = s * NC + c
        ibase = sample_base * F + w * (SP_STRIPS * 128)
        obase = w * (SP_STRIPS * 128)

        def drain(sem, dst):
            pltpu.make_async_copy(tsp.at[pl.ds(0, 128)], dst, sem).wait()

        pltpu.sync_copy(
            idx1.at[pl.ds(pl.multiple_of(ibase, 128), SP_STRIPS * 128)], sidx)

        def gath(t, dst, sem):
            gi = sidx.at[pl.ds(pl.multiple_of(t * 128, 128), 128)]
            pltpu.async_copy(tsp.at[gi], dst, sem)

        def store(t, src, sem):
            pltpu.async_copy(
                src, out.at[pl.ds(pl.multiple_of(obase + t * 128, 128), 128)], sem)

        _ring_phase(SP_STRIPS, (r0, r1, r2, r3), (g0, g1, g2, g3),
                    (s0, s1, s2, s3), gath, store, drain)
    return _sparse_body


def _seq_body(hist1, tsq, qdst2, out, hidx, qdstv, r0, r1, r2, r3, acc,
              shacc, g0, g1, g2, g3, s0, s1, s2, s3):
    c = lax.axis_index("c")
    s = lax.axis_index("s")
    w = s * NC + c

    def drain(sem, dst):
        pltpu.make_async_copy(tsq.at[pl.ds(0, 128)], dst, sem).wait()

    st0 = pltpu.async_copy(
        hist1.at[pl.ds(pl.multiple_of(w * (SQ_STRIPS * 128), 128), SQ_STRIPS * 128)],
        hidx, g0)
    st1 = pltpu.async_copy(qdst2.at[pl.ds(pl.multiple_of(w * SQ_PAD, 8), SQ_PAD)], qdstv, g1)

    def _zero(r, carry):
        for q in range(NQ):
            acc[r, pl.ds(q * 16, 16)] = jnp.zeros((16,), jnp.float32)
        return carry
    lax.fori_loop(0, BPW, _zero, 0)
    st0.wait()
    st1.wait()
    pltpu.sync_copy(acc, shacc.at[pl.ds(s * BPW, BPW)])

    def gath(t, dst, sem):
        gi = hidx.at[pl.ds(pl.multiple_of(t * 128, 128), 128)]
        pltpu.async_copy(tsq.at[gi], dst, sem)

    def store(t, src, sem):
        pltpu.async_copy(src, shacc.at[qdstv.at[t]], sem, add=True)

    _ring_phase(SQ_STRIPS, (r0, r1, r2, r3), (g0, g1, g2, g3),
                (s0, s1, s2, s3), gath, store, drain)

    pltpu.sync_copy(shacc.at[pl.ds(s * BPW, BPW)], acc)

    def _scale(r, carry):
        for q in range(NQ):
            acc[r, pl.ds(q * 16, 16)] = acc[r, pl.ds(q * 16, 16)] * (1.0 / L)
        return carry
    lax.fori_loop(0, BPW, _scale, 0)
    pltpu.sync_copy(acc, out.at[pl.ds(pl.multiple_of(w * BPW, 128), BPW)])


def _make_sparse_call(sample_base):
    return functools.partial(
        pl.kernel,
        out_type=jax.ShapeDtypeStruct((BH * F, D), jnp.float32),
        mesh=_MESH,
        compiler_params=_PARAMS,
        scratch_types=(
            [pltpu.VMEM((SP_STRIPS * 128,), jnp.int32)]
            + [pltpu.VMEM((128, D), jnp.float32)] * NBUF
            + [pltpu.SemaphoreType.DMA] * (2 * NBUF)
        ),
    )(_make_sparse_body(sample_base))


_sparse_call_0 = _make_sparse_call(0)
_sparse_call_1 = _make_sparse_call(BH)

_seq_call = functools.partial(
    pl.kernel,
    out_type=jax.ShapeDtypeStruct((B, D), jnp.float32),
    mesh=_MESH,
    compiler_params=_PARAMS,
    scratch_types=(
        [
            pltpu.VMEM((SQ_STRIPS * 128,), jnp.int32),
            pltpu.VMEM((SQ_PAD, 128), jnp.int32),
        ]
        + [pltpu.VMEM((128, D), jnp.float32)] * NBUF
        + [
            pltpu.VMEM((BPW, D), jnp.float32),
            pltpu.VMEM_SHARED((NS * BPW, D), jnp.float32),
        ]
        + [pltpu.SemaphoreType.DMA] * (2 * NBUF)
    ),
)(_seq_body)


def kernel(indices, hist, table_sparse, table_seq):
    idx1 = indices.astype(jnp.int32).reshape(-1)
    hist1 = hist.astype(jnp.int32).reshape(-1)
    sp1 = _sparse_call_0(idx1, table_sparse)
    sp2 = _sparse_call_1(idx1, table_sparse)
    pool = _seq_call(hist1, table_seq, jnp.asarray(_qdst_array()))
    sp = jnp.concatenate(
        [sp1.reshape(BH, F * D), sp2.reshape(BH, F * D)], axis=0)
    return jnp.concatenate([sp, pool], axis=-1)
